# baseline (device time: 27075 ns/iter reference)
import jax
import jax.numpy as jnp
from jax import lax
from jax.experimental import pallas as pl
from jax.experimental.pallas import tpu as pltpu

N_LAYERS = 3


def kernel(x, Win0, Wout0, Win1, Wout1, Win2, Wout2):
    b, dy = x.shape
    _, hx = Win0.shape

    def body(x_ref, win0_ref, wout0_ref, win1_ref, wout1_ref, win2_ref,
             wout2_ref, out_ref, h_send, h_recv, x_send, x_recv,
             send_sems, recv_sems):
        mx = lax.axis_index("x")
        my = lax.axis_index("y")
        y_partner = (mx, 1 - my)
        x_partner = (1 - mx, my)

        barrier = pltpu.get_barrier_semaphore()
        for nbr in (y_partner, x_partner):
            pl.semaphore_signal(
                barrier, inc=1, device_id=nbr,
                device_id_type=pl.DeviceIdType.MESH,
            )
        pl.semaphore_wait(barrier, 2)

        wins = [win0_ref, win1_ref, win2_ref]
        wouts = [wout0_ref, wout1_ref, wout2_ref]

        def exchange(send_buf, recv_buf, slot, sem_idx, partner, partial):
            send_buf[slot] = partial
            rdma = pltpu.make_async_remote_copy(
                src_ref=send_buf.at[slot],
                dst_ref=recv_buf.at[slot],
                send_sem=send_sems.at[sem_idx],
                recv_sem=recv_sems.at[sem_idx],
                device_id=partner,
                device_id_type=pl.DeviceIdType.MESH,
            )
            rdma.start()
            rdma.wait()
            return partial + recv_buf[slot]

        cur = x_ref[...]
        for l in range(N_LAYERS):
            p = jnp.dot(
                cur.astype(jnp.bfloat16),
                wins[l][...].astype(jnp.bfloat16),
                preferred_element_type=jnp.float32,
            )
            h = jnp.maximum(exchange(h_send, h_recv, l, 2 * l, y_partner, p), 0.0)
            q = jnp.dot(
                h.astype(jnp.bfloat16),
                wouts[l][...].astype(jnp.bfloat16),
                preferred_element_type=jnp.float32,
            )
            cur = exchange(x_send, x_recv, l, 2 * l + 1, x_partner, q)
        out_ref[...] = cur

    return pl.pallas_call(
        body,
        out_shape=jax.ShapeDtypeStruct((b, dy), jnp.float32),
        in_specs=[pl.BlockSpec(memory_space=pltpu.VMEM)] * 7,
        out_specs=pl.BlockSpec(memory_space=pltpu.VMEM),
        scratch_shapes=[
            pltpu.VMEM((N_LAYERS, b, hx), jnp.float32),
            pltpu.VMEM((N_LAYERS, b, hx), jnp.float32),
            pltpu.VMEM((N_LAYERS, b, dy), jnp.float32),
            pltpu.VMEM((N_LAYERS, b, dy), jnp.float32),
            pltpu.SemaphoreType.DMA((2 * N_LAYERS,)),
            pltpu.SemaphoreType.DMA((2 * N_LAYERS,)),
        ],
        compiler_params=pltpu.CompilerParams(collective_id=0),
    )(x, Win0, Wout0, Win1, Wout1, Win2, Wout2)


# device time: 10588 ns/iter; 2.5571x vs baseline; 2.5571x over previous
import jax
import jax.numpy as jnp
from jax import lax
from jax.experimental import pallas as pl
from jax.experimental.pallas import tpu as pltpu

N_LAYERS = 3


def kernel(x, Win0, Wout0, Win1, Wout1, Win2, Wout2):
    b, dy = x.shape
    _, hx = Win0.shape

    def body(x_ref, win0_ref, wout0_ref, win1_ref, wout1_ref, win2_ref,
             wout2_ref, out_ref, h_send, h_recv, x_send, x_recv,
             send_sems, recv_sems):
        mx = lax.axis_index("x")
        my = lax.axis_index("y")
        y_partner = (mx, 1 - my)
        x_partner = (1 - mx, my)

        barrier = pltpu.get_barrier_semaphore()
        for nbr in (y_partner, x_partner):
            pl.semaphore_signal(
                barrier, inc=1, device_id=nbr,
                device_id_type=pl.DeviceIdType.MESH,
            )
        pl.semaphore_wait(barrier, 2)

        wins = [win0_ref, win1_ref, win2_ref]
        wouts = [wout0_ref, wout1_ref, wout2_ref]
        inflight = []

        def exchange(send_buf, recv_buf, slot, sem_idx, partner, partial):
            send_buf[slot] = partial.astype(jnp.bfloat16)
            rdma = pltpu.make_async_remote_copy(
                src_ref=send_buf.at[slot],
                dst_ref=recv_buf.at[slot],
                send_sem=send_sems.at[sem_idx],
                recv_sem=recv_sems.at[sem_idx],
                device_id=partner,
                device_id_type=pl.DeviceIdType.MESH,
            )
            rdma.start()
            inflight.append(rdma)
            rdma.wait_recv()
            return partial + recv_buf[slot].astype(jnp.float32)

        cur = x_ref[...]
        for l in range(N_LAYERS):
            p = jnp.dot(
                cur.astype(jnp.bfloat16),
                wins[l][...].astype(jnp.bfloat16),
                preferred_element_type=jnp.float32,
            )
            h = jnp.maximum(exchange(h_send, h_recv, l, 2 * l, y_partner, p), 0.0)
            q = jnp.dot(
                h.astype(jnp.bfloat16),
                wouts[l][...].astype(jnp.bfloat16),
                preferred_element_type=jnp.float32,
            )
            cur = exchange(x_send, x_recv, l, 2 * l + 1, x_partner, q)
        out_ref[...] = cur
        for rdma in inflight:
            rdma.wait_send()

    return pl.pallas_call(
        body,
        out_shape=jax.ShapeDtypeStruct((b, dy), jnp.float32),
        in_specs=[pl.BlockSpec(memory_space=pltpu.VMEM)] * 7,
        out_specs=pl.BlockSpec(memory_space=pltpu.VMEM),
        scratch_shapes=[
            pltpu.VMEM((N_LAYERS, b, hx), jnp.bfloat16),
            pltpu.VMEM((N_LAYERS, b, hx), jnp.bfloat16),
            pltpu.VMEM((N_LAYERS, b, dy), jnp.bfloat16),
            pltpu.VMEM((N_LAYERS, b, dy), jnp.bfloat16),
            pltpu.SemaphoreType.DMA((2 * N_LAYERS,)),
            pltpu.SemaphoreType.DMA((2 * N_LAYERS,)),
        ],
        compiler_params=pltpu.CompilerParams(collective_id=0),
    )(x, Win0, Wout0, Win1, Wout1, Win2, Wout2)
